# baseline (speedup 1.0000x reference)
import functools

import jax
import jax.numpy as jnp
from jax import lax
from jax.experimental import pallas as pl
from jax.experimental.pallas import tpu as pltpu
from jax.experimental.pallas import tpu_sc as plsc

_NUM_CLASSES = 1000000
_EMB_DIM = 32
_BATCH = 16384
_G = 16


@functools.cache
def _build():
    info = plsc.get_sparse_core_info()
    num_workers = info.num_cores * info.num_subcores
    b_per_w = _BATCH // num_workers
    n_groups = b_per_w // _G
    mesh = plsc.VectorSubcoreMesh(core_axis_name="c", subcore_axis_name="s")

    @functools.partial(
        pl.kernel,
        mesh=mesh,
        out_type=jax.ShapeDtypeStruct((_EMB_DIM, _BATCH), jnp.float32),
        scratch_types=[
            pltpu.VMEM((b_per_w,), jnp.int32),
            [pltpu.VMEM((_EMB_DIM, 128), jnp.float32) for _ in range(_G)],
            pltpu.VMEM((_EMB_DIM, b_per_w), jnp.float32),
            [pltpu.SemaphoreType.DMA for _ in range(_G)],
        ],
        compiler_params=pltpu.CompilerParams(
            disable_bounds_checks=True, needs_layout_passes=False),
    )
    def emb_lookup(idx_hbm, table_t_hbm, out_t_hbm,
                   idx_v, bufs, stage_v, sems):
        wid = lax.axis_index("s") * info.num_cores + lax.axis_index("c")
        base = wid * b_per_w
        pltpu.sync_copy(idx_hbm.at[pl.ds(base, b_per_w)], idx_v)

        d_lo = lax.iota(jnp.int32, 16)
        d_hi = d_lo + 16
        lane_ids = lax.iota(jnp.int32, 16)

        def extract(vec, r):
            return jnp.max(jnp.where(lane_ids == r, vec, 0))

        def fetch_group(g):
            cvec = idx_v[pl.ds(g * _G, _G)]
            col0vec = lax.shift_left(lax.shift_right_logical(cvec, 7), 7)
            for r in range(_G):
                col0 = pl.multiple_of(extract(col0vec, r), 128)
                pltpu.async_copy(
                    table_t_hbm.at[:, pl.ds(col0, 128)], bufs[r], sems[r])

        fetch_group(0)

        def body(g, carry):
            cvec = idx_v[pl.ds(g * _G, _G)]
            lanevec = lax.bitwise_and(cvec, 127)
            for r in range(_G):
                pltpu.make_async_copy(
                    table_t_hbm.at[:, pl.ds(0, 128)], bufs[r], sems[r]).wait()
                lane = jnp.broadcast_to(extract(lanevec, r), (16,))
                pos = jnp.broadcast_to(g * _G + r, (16,))
                plsc.store_scatter(
                    stage_v, [d_lo, pos],
                    plsc.load_gather(bufs[r], [d_lo, lane]))
                plsc.store_scatter(
                    stage_v, [d_hi, pos],
                    plsc.load_gather(bufs[r], [d_hi, lane]))

            @pl.when(g + 1 < n_groups)
            def _():
                fetch_group(g + 1)
            return carry

        lax.fori_loop(0, n_groups, body, 0)
        pltpu.sync_copy(stage_v, out_t_hbm.at[:, pl.ds(base, b_per_w)])

    return emb_lookup


def kernel(condition, embedding_weight):
    out_t = _build()(condition.astype(jnp.int32), embedding_weight.T)
    return out_t.T
